# Initial kernel scaffold; baseline (speedup 1.0000x reference)
import jax
import jax.numpy as jnp
from jax.experimental import pallas as pl


def kernel(depth, covariance, rotation, opacity, sh_color, confidence, poses, img_h, img_w):
    raise NotImplementedError("write your pallas kernel here")


# speedup vs baseline: 2.6386x; 2.6386x over previous
import functools

import jax
import jax.numpy as jnp
from jax.experimental import pallas as pl
from jax.experimental.pallas import tpu as pltpu

_CONF_THRESH = 0.1
_OPACITY_THRESH = 0.01
_TH = 64


def _fusion_kernel(poses_ref, depth_ref, cov_ref, rot_ref, opac_ref, sh_ref,
                   conf_ref, fused_ref, mask_ref, *, H, W, C):
    b = pl.program_id(0)
    n = pl.program_id(1)
    hi = pl.program_id(2)

    row = jax.lax.broadcasted_iota(jnp.int32, (_TH, W), 0).astype(jnp.float32)
    col = jax.lax.broadcasted_iota(jnp.int32, (_TH, W), 1).astype(jnp.float32)
    row = row + (hi * _TH).astype(jnp.float32)
    pi = jnp.float32(jnp.pi)
    theta = (col + 0.5) * (2.0 * pi / W) - pi
    phi = (row + 0.5) * (pi / H) - pi / 2.0
    cphi = jnp.cos(phi)
    sphi = jnp.sin(phi)
    cth = jnp.cos(theta)
    sth = jnp.sin(theta)

    dep = depth_ref[0, 0]
    c0 = dep * (cphi * sth)
    c1 = dep * sphi
    c2 = dep * (cphi * cth)

    def p(i, j):
        return poses_ref[b, n, i, j]

    world = []
    for i in range(3):
        ti = -(p(0, i) * p(0, 3) + p(1, i) * p(1, 3) + p(2, i) * p(2, 3))
        world.append(p(0, i) * c0 + p(1, i) * c1 + p(2, i) * c2 + ti)

    scales = [cov_ref[0, 0, k] for k in range(3)]

    r = [rot_ref[0, 0, k] for k in range(4)]
    norm = jnp.sqrt(r[0] * r[0] + r[1] * r[1] + r[2] * r[2] + r[3] * r[3])
    inv_norm = 1.0 / (norm + 1e-8)
    rots = [rk * inv_norm for rk in r]

    opac = opac_ref[0, 0]
    conf = conf_ref[0, 0]
    shs = [sh_ref[0, 0, k] for k in range(C)]

    planes = world + scales + rots + [opac, conf] + shs
    for h in range(_TH):
        a = jnp.concatenate([pln[h:h + 1, :] for pln in planes], axis=0)
        fused_ref[0, 0, h] = a.T
    mask_ref[0, 0] = (conf > _CONF_THRESH) & (opac > _OPACITY_THRESH)


def kernel(depth, covariance, rotation, opacity, sh_color, confidence, poses,
           img_h, img_w):
    B, N, H, W = depth.shape
    C = sh_color.shape[2]
    HB = H // _TH
    NB = N * HB
    grid = (B, N, HB)

    fused5, mask4 = pl.pallas_call(
        functools.partial(_fusion_kernel, H=H, W=W, C=C),
        grid=grid,
        in_specs=[
            pl.BlockSpec(memory_space=pltpu.SMEM),
            pl.BlockSpec((1, 1, _TH, W), lambda b, n, h: (b, n, h, 0)),
            pl.BlockSpec((1, 1, 3, _TH, W), lambda b, n, h: (b, n, 0, h, 0)),
            pl.BlockSpec((1, 1, 4, _TH, W), lambda b, n, h: (b, n, 0, h, 0)),
            pl.BlockSpec((1, 1, _TH, W), lambda b, n, h: (b, n, h, 0)),
            pl.BlockSpec((1, 1, C, _TH, W), lambda b, n, h: (b, n, 0, h, 0)),
            pl.BlockSpec((1, 1, _TH, W), lambda b, n, h: (b, n, h, 0)),
        ],
        out_specs=[
            pl.BlockSpec((1, 1, _TH, W, 24),
                         lambda b, n, h: (b, n * HB + h, 0, 0, 0)),
            pl.BlockSpec((1, 1, _TH, W),
                         lambda b, n, h: (b, n * HB + h, 0, 0)),
        ],
        out_shape=[
            jax.ShapeDtypeStruct((B, NB, _TH, W, 24), jnp.float32),
            jax.ShapeDtypeStruct((B, NB, _TH, W), jnp.bool_),
        ],
    )(poses, depth, covariance, rotation, opacity, sh_color, confidence)

    P = N * H * W
    return fused5.reshape(B, P, 24), mask4.reshape(B, P)


# speedup vs baseline: 7.0680x; 2.6787x over previous
import functools

import jax
import jax.numpy as jnp
from jax.experimental import pallas as pl
from jax.experimental.pallas import tpu as pltpu

_CONF_THRESH = 0.1
_OPACITY_THRESH = 0.01
_TH = 64


def _fusion_kernel(poses_ref, depth_ref, cov_ref, rot_ref, opac_ref, sh_ref,
                   conf_ref, fused_ref, mask_ref, *, H, W, C):
    b = pl.program_id(0)
    n = pl.program_id(1)
    hi = pl.program_id(2)

    row = jax.lax.broadcasted_iota(jnp.int32, (_TH, W), 0).astype(jnp.float32)
    col = jax.lax.broadcasted_iota(jnp.int32, (_TH, W), 1).astype(jnp.float32)
    row = row + (hi * _TH).astype(jnp.float32)
    pi = jnp.float32(jnp.pi)
    theta = (col + 0.5) * (2.0 * pi / W) - pi
    phi = (row + 0.5) * (pi / H) - pi / 2.0
    cphi = jnp.cos(phi)
    sphi = jnp.sin(phi)
    cth = jnp.cos(theta)
    sth = jnp.sin(theta)

    dep = depth_ref[0, 0]
    c0 = dep * (cphi * sth)
    c1 = dep * sphi
    c2 = dep * (cphi * cth)

    def p(i, j):
        return poses_ref[b, n, i, j]

    world = []
    for i in range(3):
        ti = -(p(0, i) * p(0, 3) + p(1, i) * p(1, 3) + p(2, i) * p(2, 3))
        world.append(p(0, i) * c0 + p(1, i) * c1 + p(2, i) * c2 + ti)

    scales = [cov_ref[0, 0, k] for k in range(3)]

    r = [rot_ref[0, 0, k] for k in range(4)]
    norm = jnp.sqrt(r[0] * r[0] + r[1] * r[1] + r[2] * r[2] + r[3] * r[3])
    inv_norm = 1.0 / (norm + 1e-8)
    rots = [rk * inv_norm for rk in r]

    opac = opac_ref[0, 0]
    conf = conf_ref[0, 0]
    shs = [sh_ref[0, 0, k] for k in range(C)]

    planes = world + scales + rots + [opac, conf] + shs
    for c, pln in enumerate(planes):
        fused_ref[0, c, 0] = pln
    mask_ref[0, 0] = (conf > _CONF_THRESH) & (opac > _OPACITY_THRESH)


def kernel(depth, covariance, rotation, opacity, sh_color, confidence, poses,
           img_h, img_w):
    B, N, H, W = depth.shape
    C = sh_color.shape[2]
    HB = H // _TH
    NB = N * HB
    grid = (B, N, HB)

    fused5, mask4 = pl.pallas_call(
        functools.partial(_fusion_kernel, H=H, W=W, C=C),
        grid=grid,
        in_specs=[
            pl.BlockSpec(memory_space=pltpu.SMEM),
            pl.BlockSpec((1, 1, _TH, W), lambda b, n, h: (b, n, h, 0)),
            pl.BlockSpec((1, 1, 3, _TH, W), lambda b, n, h: (b, n, 0, h, 0)),
            pl.BlockSpec((1, 1, 4, _TH, W), lambda b, n, h: (b, n, 0, h, 0)),
            pl.BlockSpec((1, 1, _TH, W), lambda b, n, h: (b, n, h, 0)),
            pl.BlockSpec((1, 1, C, _TH, W), lambda b, n, h: (b, n, 0, h, 0)),
            pl.BlockSpec((1, 1, _TH, W), lambda b, n, h: (b, n, h, 0)),
        ],
        out_specs=[
            pl.BlockSpec((1, 24, 1, _TH, W),
                         lambda b, n, h: (b, 0, n * HB + h, 0, 0)),
            pl.BlockSpec((1, 1, _TH, W),
                         lambda b, n, h: (b, n * HB + h, 0, 0)),
        ],
        out_shape=[
            jax.ShapeDtypeStruct((B, 24, NB, _TH, W), jnp.float32),
            jax.ShapeDtypeStruct((B, NB, _TH, W), jnp.bool_),
        ],
    )(poses, depth, covariance, rotation, opacity, sh_color, confidence)

    P = N * H * W
    fused = jnp.transpose(fused5.reshape(B, 24, P), (0, 2, 1))
    return fused, mask4.reshape(B, P)


# speedup vs baseline: 7.0754x; 1.0010x over previous
import functools

import jax
import jax.numpy as jnp
from jax.experimental import pallas as pl
from jax.experimental.pallas import tpu as pltpu

_CONF_THRESH = 0.1
_OPACITY_THRESH = 0.01
_TH = 64


def _fusion_kernel(poses_ref, depth_ref, cov_ref, rot_ref, opac_ref, sh_ref,
                   conf_ref, fused_ref, mask_ref, *, H, W, C):
    b = pl.program_id(0)
    n = pl.program_id(1)
    hi = pl.program_id(2)

    row = jax.lax.broadcasted_iota(jnp.int32, (_TH, W), 0).astype(jnp.float32)
    col = jax.lax.broadcasted_iota(jnp.int32, (_TH, W), 1).astype(jnp.float32)
    row = row + (hi * _TH).astype(jnp.float32)
    pi = jnp.float32(jnp.pi)
    theta = (col + 0.5) * (2.0 * pi / W) - pi
    phi = (row + 0.5) * (pi / H) - pi / 2.0
    cphi = jnp.cos(phi)
    sphi = jnp.sin(phi)
    cth = jnp.cos(theta)
    sth = jnp.sin(theta)

    dep = depth_ref[0, 0]
    c0 = dep * (cphi * sth)
    c1 = dep * sphi
    c2 = dep * (cphi * cth)

    def p(i, j):
        return poses_ref[b, n, i, j]

    world = []
    for i in range(3):
        ti = -(p(0, i) * p(0, 3) + p(1, i) * p(1, 3) + p(2, i) * p(2, 3))
        world.append(p(0, i) * c0 + p(1, i) * c1 + p(2, i) * c2 + ti)

    scales = [cov_ref[0, 0, k] for k in range(3)]

    r = [rot_ref[0, 0, k] for k in range(4)]
    norm = jnp.sqrt(r[0] * r[0] + r[1] * r[1] + r[2] * r[2] + r[3] * r[3])
    inv_norm = 1.0 / (norm + 1e-8)
    rots = [rk * inv_norm for rk in r]

    opac = opac_ref[0, 0]
    conf = conf_ref[0, 0]
    shs = [sh_ref[0, 0, k] for k in range(C)]

    planes = world + scales + rots + [opac, conf] + shs
    for c, pln in enumerate(planes):
        fused_ref[0, c, 0] = pln
    mask_ref[0, 0] = ((conf > _CONF_THRESH) & (opac > _OPACITY_THRESH)).astype(jnp.int8)


def kernel(depth, covariance, rotation, opacity, sh_color, confidence, poses,
           img_h, img_w):
    B, N, H, W = depth.shape
    C = sh_color.shape[2]
    HB = H // _TH
    NB = N * HB
    grid = (B, N, HB)

    fused5, mask4 = pl.pallas_call(
        functools.partial(_fusion_kernel, H=H, W=W, C=C),
        grid=grid,
        in_specs=[
            pl.BlockSpec(memory_space=pltpu.SMEM),
            pl.BlockSpec((1, 1, _TH, W), lambda b, n, h: (b, n, h, 0)),
            pl.BlockSpec((1, 1, 3, _TH, W), lambda b, n, h: (b, n, 0, h, 0)),
            pl.BlockSpec((1, 1, 4, _TH, W), lambda b, n, h: (b, n, 0, h, 0)),
            pl.BlockSpec((1, 1, _TH, W), lambda b, n, h: (b, n, h, 0)),
            pl.BlockSpec((1, 1, C, _TH, W), lambda b, n, h: (b, n, 0, h, 0)),
            pl.BlockSpec((1, 1, _TH, W), lambda b, n, h: (b, n, h, 0)),
        ],
        out_specs=[
            pl.BlockSpec((1, 24, 1, _TH, W),
                         lambda b, n, h: (b, 0, n * HB + h, 0, 0)),
            pl.BlockSpec((1, 1, _TH, W),
                         lambda b, n, h: (b, n * HB + h, 0, 0)),
        ],
        out_shape=[
            jax.ShapeDtypeStruct((B, 24, NB, _TH, W), jnp.float32),
            jax.ShapeDtypeStruct((B, NB, _TH, W), jnp.int8),
        ],
    )(poses, depth, covariance, rotation, opacity, sh_color, confidence)

    P = N * H * W
    fused = jnp.transpose(fused5.reshape(B, 24, P), (0, 2, 1))
    return fused, mask4.reshape(B, P).astype(jnp.bool_)


# speedup vs baseline: 9.7793x; 1.3822x over previous
import functools

import jax
import jax.numpy as jnp
from jax.experimental import pallas as pl
from jax.experimental.pallas import tpu as pltpu

_CONF_THRESH = 0.1
_OPACITY_THRESH = 0.01
_TH = 64


def _fusion_kernel(poses_ref, depth_ref, cov_ref, rot_ref, opac_ref, sh_ref,
                   conf_ref, fused_ref, mask_ref, *, H, W, C):
    b = pl.program_id(0)
    n = pl.program_id(1)
    hi = pl.program_id(2)

    row = jax.lax.broadcasted_iota(jnp.int32, (_TH, W), 0).astype(jnp.float32)
    col = jax.lax.broadcasted_iota(jnp.int32, (_TH, W), 1).astype(jnp.float32)
    row = row + (hi * _TH).astype(jnp.float32)
    pi = jnp.float32(jnp.pi)
    theta = (col + 0.5) * (2.0 * pi / W) - pi
    phi = (row + 0.5) * (pi / H) - pi / 2.0
    cphi = jnp.cos(phi)
    sphi = jnp.sin(phi)
    cth = jnp.cos(theta)
    sth = jnp.sin(theta)

    dep = depth_ref[0, 0]
    c0 = dep * (cphi * sth)
    c1 = dep * sphi
    c2 = dep * (cphi * cth)

    def p(i, j):
        return poses_ref[b, n, i, j]

    world = []
    for i in range(3):
        ti = -(p(0, i) * p(0, 3) + p(1, i) * p(1, 3) + p(2, i) * p(2, 3))
        world.append(p(0, i) * c0 + p(1, i) * c1 + p(2, i) * c2 + ti)

    scales = [cov_ref[0, 0, k] for k in range(3)]

    r = [rot_ref[0, 0, k] for k in range(4)]
    norm = jnp.sqrt(r[0] * r[0] + r[1] * r[1] + r[2] * r[2] + r[3] * r[3])
    inv_norm = 1.0 / (norm + 1e-8)
    rots = [rk * inv_norm for rk in r]

    opac = opac_ref[0, 0]
    conf = conf_ref[0, 0]
    shs = [sh_ref[0, 0, k] for k in range(C)]

    planes = world + scales + rots + [opac, conf] + shs
    for c, pln in enumerate(planes):
        for w2 in range(W // 128):
            fused_ref[0, c // 8, 0, :, w2, c % 8, :] = (
                pln[:, 128 * w2:128 * (w2 + 1)])
    mask_ref[0, 0] = ((conf > _CONF_THRESH) & (opac > _OPACITY_THRESH)).astype(jnp.int8)


def kernel(depth, covariance, rotation, opacity, sh_color, confidence, poses,
           img_h, img_w):
    B, N, H, W = depth.shape
    C = sh_color.shape[2]
    HB = H // _TH
    NB = N * HB
    grid = (B, N, HB)
    P = N * H * W
    TPB = _TH * W // 128

    fused5, mask4 = pl.pallas_call(
        functools.partial(_fusion_kernel, H=H, W=W, C=C),
        grid=grid,
        in_specs=[
            pl.BlockSpec(memory_space=pltpu.SMEM),
            pl.BlockSpec((1, 1, _TH, W), lambda b, n, h: (b, n, h, 0)),
            pl.BlockSpec((1, 1, 3, _TH, W), lambda b, n, h: (b, n, 0, h, 0)),
            pl.BlockSpec((1, 1, 4, _TH, W), lambda b, n, h: (b, n, 0, h, 0)),
            pl.BlockSpec((1, 1, _TH, W), lambda b, n, h: (b, n, h, 0)),
            pl.BlockSpec((1, 1, C, _TH, W), lambda b, n, h: (b, n, 0, h, 0)),
            pl.BlockSpec((1, 1, _TH, W), lambda b, n, h: (b, n, h, 0)),
        ],
        out_specs=[
            pl.BlockSpec((1, 3, 1, _TH, W // 128, 8, 128),
                         lambda b, n, h: (b, 0, n * HB + h, 0, 0, 0, 0)),
            pl.BlockSpec((1, 1, _TH, W),
                         lambda b, n, h: (b, n * HB + h, 0, 0)),
        ],
        out_shape=[
            jax.ShapeDtypeStruct((B, 3, NB, _TH, W // 128, 8, 128),
                                 jnp.float32),
            jax.ShapeDtypeStruct((B, NB, _TH, W), jnp.int8),
        ],
    )(poses, depth, covariance, rotation, opacity, sh_color, confidence)

    fused = jnp.transpose(fused5.reshape(B, 3, P // 128, 8, 128),
                          (0, 2, 4, 1, 3)).reshape(B, P, 24)
    return fused, mask4.reshape(B, P).astype(jnp.bool_)


# speedup vs baseline: 9.8300x; 1.0052x over previous
import functools

import jax
import jax.numpy as jnp
from jax.experimental import pallas as pl
from jax.experimental.pallas import tpu as pltpu

_CONF_THRESH = 0.1
_OPACITY_THRESH = 0.01
_TH = 64


def _fusion_kernel(poses_ref, depth_ref, cov_ref, rot_ref, opac_ref, sh_ref,
                   conf_ref, fused_ref, mask_ref, *, H, W, C):
    b = pl.program_id(0)
    n = pl.program_id(1)
    hi = pl.program_id(2)

    row = jax.lax.broadcasted_iota(jnp.int32, (_TH, W), 0).astype(jnp.float32)
    col = jax.lax.broadcasted_iota(jnp.int32, (_TH, W), 1).astype(jnp.float32)
    row = row + (hi * _TH).astype(jnp.float32)
    pi = jnp.float32(jnp.pi)
    theta = (col + 0.5) * (2.0 * pi / W) - pi
    phi = (row + 0.5) * (pi / H) - pi / 2.0
    cphi = jnp.cos(phi)
    sphi = jnp.sin(phi)
    cth = jnp.cos(theta)
    sth = jnp.sin(theta)

    dep = depth_ref[0, 0]
    c0 = dep * (cphi * sth)
    c1 = dep * sphi
    c2 = dep * (cphi * cth)

    def p(i, j):
        return poses_ref[b, n, i, j]

    world = []
    for i in range(3):
        ti = -(p(0, i) * p(0, 3) + p(1, i) * p(1, 3) + p(2, i) * p(2, 3))
        world.append(p(0, i) * c0 + p(1, i) * c1 + p(2, i) * c2 + ti)

    scales = [cov_ref[0, 0, k] for k in range(3)]

    r = [rot_ref[0, 0, k] for k in range(4)]
    norm = jnp.sqrt(r[0] * r[0] + r[1] * r[1] + r[2] * r[2] + r[3] * r[3])
    inv_norm = 1.0 / (norm + 1e-8)
    rots = [rk * inv_norm for rk in r]

    opac = opac_ref[0, 0]
    conf = conf_ref[0, 0]
    shs = [sh_ref[0, 0, k] for k in range(C)]

    planes = world + scales + rots + [opac, conf] + shs
    for c, pln in enumerate(planes):
        for w2 in range(W // 128):
            fused_ref[0, c // 8, 0, :, w2, c % 8, :] = (
                pln[:, 128 * w2:128 * (w2 + 1)])
    mask_ref[0, 0] = ((conf > _CONF_THRESH) & (opac > _OPACITY_THRESH)).astype(jnp.int8)


def kernel(depth, covariance, rotation, opacity, sh_color, confidence, poses,
           img_h, img_w):
    B, N, H, W = depth.shape
    C = sh_color.shape[2]
    HB = H // _TH
    NB = N * HB
    grid = (B, N, HB)
    P = N * H * W
    TPB = _TH * W // 128

    fused5, mask4 = pl.pallas_call(
        functools.partial(_fusion_kernel, H=H, W=W, C=C),
        grid=grid,
        in_specs=[
            pl.BlockSpec(memory_space=pltpu.SMEM),
            pl.BlockSpec((1, 1, _TH, W), lambda b, n, h: (b, n, h, 0)),
            pl.BlockSpec((1, 1, 3, _TH, W), lambda b, n, h: (b, n, 0, h, 0)),
            pl.BlockSpec((1, 1, 4, _TH, W), lambda b, n, h: (b, n, 0, h, 0)),
            pl.BlockSpec((1, 1, _TH, W), lambda b, n, h: (b, n, h, 0)),
            pl.BlockSpec((1, 1, C, _TH, W), lambda b, n, h: (b, n, 0, h, 0)),
            pl.BlockSpec((1, 1, _TH, W), lambda b, n, h: (b, n, h, 0)),
        ],
        out_specs=[
            pl.BlockSpec((1, 3, 1, _TH, W // 128, 8, 128),
                         lambda b, n, h: (b, 0, n * HB + h, 0, 0, 0, 0)),
            pl.BlockSpec((1, 1, _TH, W),
                         lambda b, n, h: (b, n * HB + h, 0, 0)),
        ],
        out_shape=[
            jax.ShapeDtypeStruct((B, 3, NB, _TH, W // 128, 8, 128),
                                 jnp.float32),
            jax.ShapeDtypeStruct((B, NB, _TH, W), jnp.int8),
        ],
        compiler_params=pltpu.CompilerParams(
            dimension_semantics=("parallel", "parallel", "parallel")),
    )(poses, depth, covariance, rotation, opacity, sh_color, confidence)

    fused = jnp.transpose(fused5.reshape(B, 3, P // 128, 8, 128),
                          (0, 2, 4, 1, 3)).reshape(B, P, 24)
    return fused, mask4.reshape(B, P).astype(jnp.bool_)


# speedup vs baseline: 11.2206x; 1.1415x over previous
import functools

import jax
import jax.numpy as jnp
from jax.experimental import pallas as pl
from jax.experimental.pallas import tpu as pltpu

_CONF_THRESH = 0.1
_OPACITY_THRESH = 0.01
_TH = 64


def _fusion_kernel(poses_ref, depth_ref, cov_ref, rot_ref, opac_ref, sh_ref,
                   conf_ref, fused_ref, mask_ref, *, H, W, C):
    b = pl.program_id(0)
    n = pl.program_id(1)
    hi = pl.program_id(2)

    row = jax.lax.broadcasted_iota(jnp.int32, (_TH, W), 0).astype(jnp.float32)
    col = jax.lax.broadcasted_iota(jnp.int32, (_TH, W), 1).astype(jnp.float32)
    row = row + (hi * _TH).astype(jnp.float32)
    pi = jnp.float32(jnp.pi)
    theta = (col + 0.5) * (2.0 * pi / W) - pi
    phi = (row + 0.5) * (pi / H) - pi / 2.0
    cphi = jnp.cos(phi)
    sphi = jnp.sin(phi)
    cth = jnp.cos(theta)
    sth = jnp.sin(theta)

    dep = depth_ref[0, 0]
    c0 = dep * (cphi * sth)
    c1 = dep * sphi
    c2 = dep * (cphi * cth)

    def p(i, j):
        return poses_ref[b, n, i, j]

    world = []
    for i in range(3):
        ti = -(p(0, i) * p(0, 3) + p(1, i) * p(1, 3) + p(2, i) * p(2, 3))
        world.append(p(0, i) * c0 + p(1, i) * c1 + p(2, i) * c2 + ti)

    scales = [cov_ref[0, 0, k] for k in range(3)]

    r = [rot_ref[0, 0, k] for k in range(4)]
    norm = jnp.sqrt(r[0] * r[0] + r[1] * r[1] + r[2] * r[2] + r[3] * r[3])
    inv_norm = 1.0 / (norm + 1e-8)
    rots = [rk * inv_norm for rk in r]

    opac = opac_ref[0, 0]
    conf = conf_ref[0, 0]
    shs = [sh_ref[0, 0, k] for k in range(C)]

    planes = world + scales + rots + [opac, conf] + shs
    for g in range(3):
        q = jnp.stack([planes[8 * g + s].reshape(2 * _TH, 128)
                       for s in range(8)], axis=0)
        fused_ref[0, g, 0] = jnp.swapaxes(q, 0, 1).reshape(
            _TH, W // 128, 8, 128)
    mask_ref[0, 0] = ((conf > _CONF_THRESH) & (opac > _OPACITY_THRESH)).astype(jnp.int8)


def kernel(depth, covariance, rotation, opacity, sh_color, confidence, poses,
           img_h, img_w):
    B, N, H, W = depth.shape
    C = sh_color.shape[2]
    HB = H // _TH
    NB = N * HB
    grid = (B, N, HB)
    P = N * H * W
    TPB = _TH * W // 128

    fused5, mask4 = pl.pallas_call(
        functools.partial(_fusion_kernel, H=H, W=W, C=C),
        grid=grid,
        in_specs=[
            pl.BlockSpec(memory_space=pltpu.SMEM),
            pl.BlockSpec((1, 1, _TH, W), lambda b, n, h: (b, n, h, 0)),
            pl.BlockSpec((1, 1, 3, _TH, W), lambda b, n, h: (b, n, 0, h, 0)),
            pl.BlockSpec((1, 1, 4, _TH, W), lambda b, n, h: (b, n, 0, h, 0)),
            pl.BlockSpec((1, 1, _TH, W), lambda b, n, h: (b, n, h, 0)),
            pl.BlockSpec((1, 1, C, _TH, W), lambda b, n, h: (b, n, 0, h, 0)),
            pl.BlockSpec((1, 1, _TH, W), lambda b, n, h: (b, n, h, 0)),
        ],
        out_specs=[
            pl.BlockSpec((1, 3, 1, _TH, W // 128, 8, 128),
                         lambda b, n, h: (b, 0, n * HB + h, 0, 0, 0, 0)),
            pl.BlockSpec((1, 1, _TH, W),
                         lambda b, n, h: (b, n * HB + h, 0, 0)),
        ],
        out_shape=[
            jax.ShapeDtypeStruct((B, 3, NB, _TH, W // 128, 8, 128),
                                 jnp.float32),
            jax.ShapeDtypeStruct((B, NB, _TH, W), jnp.int8),
        ],
        compiler_params=pltpu.CompilerParams(
            dimension_semantics=("parallel", "parallel", "parallel")),
    )(poses, depth, covariance, rotation, opacity, sh_color, confidence)

    fused = jnp.transpose(fused5.reshape(B, 3, P // 128, 8, 128),
                          (0, 2, 4, 1, 3)).reshape(B, P, 24)
    return fused, mask4.reshape(B, P).astype(jnp.bool_)


# speedup vs baseline: 13.1473x; 1.1717x over previous
import functools

import jax
import jax.numpy as jnp
from jax.experimental import pallas as pl
from jax.experimental.pallas import tpu as pltpu

_CONF_THRESH = 0.1
_OPACITY_THRESH = 0.01
_TH = 128


def _fusion_kernel(poses_ref, depth_ref, cov_ref, rot_ref, opac_ref, sh_ref,
                   conf_ref, fused_ref, mask_ref, *, H, W, C):
    b = pl.program_id(0)
    n = pl.program_id(1)
    hi = pl.program_id(2)

    row = jax.lax.broadcasted_iota(jnp.int32, (_TH, W), 0).astype(jnp.float32)
    col = jax.lax.broadcasted_iota(jnp.int32, (_TH, W), 1).astype(jnp.float32)
    row = row + (hi * _TH).astype(jnp.float32)
    pi = jnp.float32(jnp.pi)
    theta = (col + 0.5) * (2.0 * pi / W) - pi
    phi = (row + 0.5) * (pi / H) - pi / 2.0
    cphi = jnp.cos(phi)
    sphi = jnp.sin(phi)
    cth = jnp.cos(theta)
    sth = jnp.sin(theta)

    dep = depth_ref[0, 0]
    c0 = dep * (cphi * sth)
    c1 = dep * sphi
    c2 = dep * (cphi * cth)

    def p(i, j):
        return poses_ref[b, n, i, j]

    world = []
    for i in range(3):
        ti = -(p(0, i) * p(0, 3) + p(1, i) * p(1, 3) + p(2, i) * p(2, 3))
        world.append(p(0, i) * c0 + p(1, i) * c1 + p(2, i) * c2 + ti)

    scales = [cov_ref[0, 0, k] for k in range(3)]

    r = [rot_ref[0, 0, k] for k in range(4)]
    norm = jnp.sqrt(r[0] * r[0] + r[1] * r[1] + r[2] * r[2] + r[3] * r[3])
    inv_norm = 1.0 / (norm + 1e-8)
    rots = [rk * inv_norm for rk in r]

    opac = opac_ref[0, 0]
    conf = conf_ref[0, 0]
    shs = [sh_ref[0, 0, k] for k in range(C)]

    planes = world + scales + rots + [opac, conf] + shs
    for g in range(3):
        q = jnp.stack([planes[8 * g + s].reshape(2 * _TH, 128)
                       for s in range(8)], axis=0)
        fused_ref[0, g, 0] = jnp.swapaxes(q, 0, 1).reshape(
            _TH, W // 128, 8, 128)
    mask_ref[0, 0] = ((conf > _CONF_THRESH) & (opac > _OPACITY_THRESH)).astype(jnp.int8)


def kernel(depth, covariance, rotation, opacity, sh_color, confidence, poses,
           img_h, img_w):
    B, N, H, W = depth.shape
    C = sh_color.shape[2]
    HB = H // _TH
    NB = N * HB
    grid = (B, N, HB)
    P = N * H * W
    TPB = _TH * W // 128

    fused5, mask4 = pl.pallas_call(
        functools.partial(_fusion_kernel, H=H, W=W, C=C),
        grid=grid,
        in_specs=[
            pl.BlockSpec(memory_space=pltpu.SMEM),
            pl.BlockSpec((1, 1, _TH, W), lambda b, n, h: (b, n, h, 0)),
            pl.BlockSpec((1, 1, 3, _TH, W), lambda b, n, h: (b, n, 0, h, 0)),
            pl.BlockSpec((1, 1, 4, _TH, W), lambda b, n, h: (b, n, 0, h, 0)),
            pl.BlockSpec((1, 1, _TH, W), lambda b, n, h: (b, n, h, 0)),
            pl.BlockSpec((1, 1, C, _TH, W), lambda b, n, h: (b, n, 0, h, 0)),
            pl.BlockSpec((1, 1, _TH, W), lambda b, n, h: (b, n, h, 0)),
        ],
        out_specs=[
            pl.BlockSpec((1, 3, 1, _TH, W // 128, 8, 128),
                         lambda b, n, h: (b, 0, n * HB + h, 0, 0, 0, 0)),
            pl.BlockSpec((1, 1, _TH, W),
                         lambda b, n, h: (b, n * HB + h, 0, 0)),
        ],
        out_shape=[
            jax.ShapeDtypeStruct((B, 3, NB, _TH, W // 128, 8, 128),
                                 jnp.float32),
            jax.ShapeDtypeStruct((B, NB, _TH, W), jnp.int8),
        ],
        compiler_params=pltpu.CompilerParams(
            dimension_semantics=("parallel", "parallel", "parallel")),
    )(poses, depth, covariance, rotation, opacity, sh_color, confidence)

    fused = jnp.transpose(fused5.reshape(B, 3, P // 128, 8, 128),
                          (0, 2, 4, 1, 3)).reshape(B, P, 24)
    return fused, mask4.reshape(B, P).astype(jnp.bool_)


# speedup vs baseline: 13.2268x; 1.0060x over previous
import functools

import jax
import jax.numpy as jnp
from jax.experimental import pallas as pl
from jax.experimental.pallas import tpu as pltpu

_CONF_THRESH = 0.1
_OPACITY_THRESH = 0.01
_TH = 128


def _fusion_kernel(poses_ref, depth_ref, cov_ref, rot_ref, opac_ref, sh_ref,
                   conf_ref, fused_ref, mask_ref, *, H, W, C):
    b = pl.program_id(0)
    n = pl.program_id(1)
    hi = pl.program_id(2)

    row = jax.lax.broadcasted_iota(jnp.int32, (_TH, W), 0).astype(jnp.float32)
    col = jax.lax.broadcasted_iota(jnp.int32, (_TH, W), 1).astype(jnp.float32)
    row = row + (hi * _TH).astype(jnp.float32)
    pi = jnp.float32(jnp.pi)
    theta = (col + 0.5) * (2.0 * pi / W) - pi
    phi = (row + 0.5) * (pi / H) - pi / 2.0
    cphi = jnp.cos(phi)
    sphi = jnp.sin(phi)
    cth = jnp.cos(theta)
    sth = jnp.sin(theta)

    dep = depth_ref[0, 0]
    c0 = dep * (cphi * sth)
    c1 = dep * sphi
    c2 = dep * (cphi * cth)

    def p(i, j):
        return poses_ref[b, n, i, j]

    world = []
    for i in range(3):
        ti = -(p(0, i) * p(0, 3) + p(1, i) * p(1, 3) + p(2, i) * p(2, 3))
        world.append(p(0, i) * c0 + p(1, i) * c1 + p(2, i) * c2 + ti)

    scales = [cov_ref[0, 0, k] for k in range(3)]

    r = [rot_ref[0, 0, k] for k in range(4)]
    norm = jnp.sqrt(r[0] * r[0] + r[1] * r[1] + r[2] * r[2] + r[3] * r[3])
    inv_norm = 1.0 / (norm + 1e-8)
    rots = [rk * inv_norm for rk in r]

    opac = opac_ref[0, 0]
    conf = conf_ref[0, 0]
    shs = [sh_ref[0, 0, k] for k in range(C)]

    planes = world + scales + rots + [opac, conf] + shs
    for g in range(3):
        q = jnp.stack([planes[8 * g + s].reshape(_TH * (W // 128), 128)
                       for s in range(8)], axis=0)
        fused_ref[0, g, 0] = jnp.swapaxes(q, 0, 1).reshape(
            _TH, W // 128, 8, 128)
    mask_ref[0, 0] = ((conf > _CONF_THRESH) & (opac > _OPACITY_THRESH)).astype(jnp.int8)


def kernel(depth, covariance, rotation, opacity, sh_color, confidence, poses,
           img_h, img_w):
    B, N, H, W = depth.shape
    C = sh_color.shape[2]
    HB = H // _TH
    NB = N * HB
    grid = (B, N, HB)
    P = N * H * W

    fused5, mask4 = pl.pallas_call(
        functools.partial(_fusion_kernel, H=H, W=W, C=C),
        grid=grid,
        in_specs=[
            pl.BlockSpec(memory_space=pltpu.SMEM),
            pl.BlockSpec((1, 1, _TH, W), lambda b, n, h: (b, n, h, 0)),
            pl.BlockSpec((1, 1, 3, _TH, W), lambda b, n, h: (b, n, 0, h, 0)),
            pl.BlockSpec((1, 1, 4, _TH, W), lambda b, n, h: (b, n, 0, h, 0)),
            pl.BlockSpec((1, 1, _TH, W), lambda b, n, h: (b, n, h, 0)),
            pl.BlockSpec((1, 1, C, _TH, W), lambda b, n, h: (b, n, 0, h, 0)),
            pl.BlockSpec((1, 1, _TH, W), lambda b, n, h: (b, n, h, 0)),
        ],
        out_specs=[
            pl.BlockSpec((1, 3, 1, _TH, W // 128, 8, 128),
                         lambda b, n, h: (b, 0, n * HB + h, 0, 0, 0, 0)),
            pl.BlockSpec((1, 1, _TH, W),
                         lambda b, n, h: (b, n * HB + h, 0, 0)),
        ],
        out_shape=[
            jax.ShapeDtypeStruct((B, 3, NB, _TH, W // 128, 8, 128),
                                 jnp.float32),
            jax.ShapeDtypeStruct((B, NB, _TH, W), jnp.int8),
        ],
        compiler_params=pltpu.CompilerParams(
            dimension_semantics=("parallel", "parallel", "parallel")),
    )(poses, depth, covariance, rotation, opacity, sh_color, confidence)

    fused = jnp.transpose(fused5.reshape(B, 3, P // 128, 8, 128),
                          (0, 2, 4, 1, 3)).reshape(B, P, 24)
    return fused, mask4.reshape(B, P).astype(jnp.bool_)


# speedup vs baseline: 13.6364x; 1.0310x over previous
import functools

import jax
import jax.numpy as jnp
from jax.experimental import pallas as pl
from jax.experimental.pallas import tpu as pltpu

_CONF_THRESH = 0.1
_OPACITY_THRESH = 0.01
_TH = 128
_NV = 2


def _fusion_kernel(poses_ref, depth_ref, cov_ref, rot_ref, opac_ref, sh_ref,
                   conf_ref, fused_ref, mask_ref, *, H, W, C):
    b = pl.program_id(0)
    ng = pl.program_id(1)
    hi = pl.program_id(2)

    row = jax.lax.broadcasted_iota(jnp.int32, (_TH, W), 0).astype(jnp.float32)
    col = jax.lax.broadcasted_iota(jnp.int32, (_TH, W), 1).astype(jnp.float32)
    row = row + (hi * _TH).astype(jnp.float32)
    pi = jnp.float32(jnp.pi)
    theta = (col + 0.5) * (2.0 * pi / W) - pi
    phi = (row + 0.5) * (pi / H) - pi / 2.0
    dx = jnp.cos(phi) * jnp.sin(theta)
    dy = jnp.sin(phi)
    dz = jnp.cos(phi) * jnp.cos(theta)

    for v in range(_NV):
        dep = depth_ref[0, v]
        c0 = dep * dx
        c1 = dep * dy
        c2 = dep * dz

        def p(i, j):
            return poses_ref[b, ng * _NV + v, i, j]

        world = []
        for i in range(3):
            ti = -(p(0, i) * p(0, 3) + p(1, i) * p(1, 3) + p(2, i) * p(2, 3))
            world.append(p(0, i) * c0 + p(1, i) * c1 + p(2, i) * c2 + ti)

        scales = [cov_ref[0, v, k] for k in range(3)]

        r = [rot_ref[0, v, k] for k in range(4)]
        norm = jnp.sqrt(r[0] * r[0] + r[1] * r[1] + r[2] * r[2] + r[3] * r[3])
        inv_norm = 1.0 / (norm + 1e-8)
        rots = [rk * inv_norm for rk in r]

        opac = opac_ref[0, v]
        conf = conf_ref[0, v]
        shs = [sh_ref[0, v, k] for k in range(C)]

        planes = world + scales + rots + [opac, conf] + shs
        for g in range(3):
            q = jnp.stack([planes[8 * g + s].reshape(_TH * (W // 128), 128)
                           for s in range(8)], axis=0)
            fused_ref[0, g, v] = jnp.swapaxes(q, 0, 1).reshape(
                _TH, W // 128, 8, 128)
        mask_ref[0, v] = ((conf > _CONF_THRESH)
                          & (opac > _OPACITY_THRESH)).astype(jnp.int8)


def kernel(depth, covariance, rotation, opacity, sh_color, confidence, poses,
           img_h, img_w):
    B, N, H, W = depth.shape
    C = sh_color.shape[2]
    HB = H // _TH
    NB = N * HB
    grid = (B, N // _NV, HB)
    P = N * H * W

    fused5, mask4 = pl.pallas_call(
        functools.partial(_fusion_kernel, H=H, W=W, C=C),
        grid=grid,
        in_specs=[
            pl.BlockSpec(memory_space=pltpu.SMEM),
            pl.BlockSpec((1, _NV, _TH, W), lambda b, n, h: (b, n, h, 0)),
            pl.BlockSpec((1, _NV, 3, _TH, W), lambda b, n, h: (b, n, 0, h, 0)),
            pl.BlockSpec((1, _NV, 4, _TH, W), lambda b, n, h: (b, n, 0, h, 0)),
            pl.BlockSpec((1, _NV, _TH, W), lambda b, n, h: (b, n, h, 0)),
            pl.BlockSpec((1, _NV, C, _TH, W), lambda b, n, h: (b, n, 0, h, 0)),
            pl.BlockSpec((1, _NV, _TH, W), lambda b, n, h: (b, n, h, 0)),
        ],
        out_specs=[
            pl.BlockSpec((1, 3, _NV * HB, _TH, W // 128, 8, 128),
                         lambda b, n, h: (b, 0, n * HB + h, 0, 0, 0, 0)),
            pl.BlockSpec((1, _NV * HB, _TH, W),
                         lambda b, n, h: (b, n * HB + h, 0, 0)),
        ],
        out_shape=[
            jax.ShapeDtypeStruct((B, 3, NB, _TH, W // 128, 8, 128),
                                 jnp.float32),
            jax.ShapeDtypeStruct((B, NB, _TH, W), jnp.int8),
        ],
        compiler_params=pltpu.CompilerParams(
            dimension_semantics=("parallel", "parallel", "parallel")),
    )(poses, depth, covariance, rotation, opacity, sh_color, confidence)

    fused = jnp.transpose(fused5.reshape(B, 3, P // 128, 8, 128),
                          (0, 2, 4, 1, 3)).reshape(B, P, 24)
    return fused, mask4.reshape(B, P).astype(jnp.bool_)


# speedup vs baseline: 15.2513x; 1.1184x over previous
import functools

import jax
import jax.numpy as jnp
from jax.experimental import pallas as pl
from jax.experimental.pallas import tpu as pltpu

_CONF_THRESH = 0.1
_OPACITY_THRESH = 0.01
_TH = 128
_NV = 4


def _fusion_kernel(poses_ref, depth_ref, cov_ref, rot_ref, opac_ref, sh_ref,
                   conf_ref, fused_ref, mask_ref, *, H, W, C):
    b = pl.program_id(0)
    ng = pl.program_id(1)
    hi = pl.program_id(2)

    row = jax.lax.broadcasted_iota(jnp.int32, (_TH, W), 0).astype(jnp.float32)
    col = jax.lax.broadcasted_iota(jnp.int32, (_TH, W), 1).astype(jnp.float32)
    row = row + (hi * _TH).astype(jnp.float32)
    pi = jnp.float32(jnp.pi)
    theta = (col + 0.5) * (2.0 * pi / W) - pi
    phi = (row + 0.5) * (pi / H) - pi / 2.0
    dx = jnp.cos(phi) * jnp.sin(theta)
    dy = jnp.sin(phi)
    dz = jnp.cos(phi) * jnp.cos(theta)

    for v in range(_NV):
        dep = depth_ref[0, v]
        c0 = dep * dx
        c1 = dep * dy
        c2 = dep * dz

        def p(i, j):
            return poses_ref[b, ng * _NV + v, i, j]

        world = []
        for i in range(3):
            ti = -(p(0, i) * p(0, 3) + p(1, i) * p(1, 3) + p(2, i) * p(2, 3))
            world.append(p(0, i) * c0 + p(1, i) * c1 + p(2, i) * c2 + ti)

        scales = [cov_ref[0, v, k] for k in range(3)]

        r = [rot_ref[0, v, k] for k in range(4)]
        norm = jnp.sqrt(r[0] * r[0] + r[1] * r[1] + r[2] * r[2] + r[3] * r[3])
        inv_norm = 1.0 / (norm + 1e-8)
        rots = [rk * inv_norm for rk in r]

        opac = opac_ref[0, v]
        conf = conf_ref[0, v]
        shs = [sh_ref[0, v, k] for k in range(C)]

        planes = world + scales + rots + [opac, conf] + shs
        for g in range(3):
            q = jnp.stack([planes[8 * g + s].reshape(_TH * (W // 128), 128)
                           for s in range(8)], axis=0)
            fused_ref[0, g, v] = jnp.swapaxes(q, 0, 1).reshape(
                _TH, W // 128, 8, 128)
        mask_ref[0, v] = ((conf > _CONF_THRESH)
                          & (opac > _OPACITY_THRESH)).astype(jnp.int8)


def kernel(depth, covariance, rotation, opacity, sh_color, confidence, poses,
           img_h, img_w):
    B, N, H, W = depth.shape
    C = sh_color.shape[2]
    HB = H // _TH
    NB = N * HB
    grid = (B, N // _NV, HB)
    P = N * H * W

    fused5, mask4 = pl.pallas_call(
        functools.partial(_fusion_kernel, H=H, W=W, C=C),
        grid=grid,
        in_specs=[
            pl.BlockSpec(memory_space=pltpu.SMEM),
            pl.BlockSpec((1, _NV, _TH, W), lambda b, n, h: (b, n, h, 0)),
            pl.BlockSpec((1, _NV, 3, _TH, W), lambda b, n, h: (b, n, 0, h, 0)),
            pl.BlockSpec((1, _NV, 4, _TH, W), lambda b, n, h: (b, n, 0, h, 0)),
            pl.BlockSpec((1, _NV, _TH, W), lambda b, n, h: (b, n, h, 0)),
            pl.BlockSpec((1, _NV, C, _TH, W), lambda b, n, h: (b, n, 0, h, 0)),
            pl.BlockSpec((1, _NV, _TH, W), lambda b, n, h: (b, n, h, 0)),
        ],
        out_specs=[
            pl.BlockSpec((1, 3, _NV * HB, _TH, W // 128, 8, 128),
                         lambda b, n, h: (b, 0, n * HB + h, 0, 0, 0, 0)),
            pl.BlockSpec((1, _NV * HB, _TH, W),
                         lambda b, n, h: (b, n * HB + h, 0, 0)),
        ],
        out_shape=[
            jax.ShapeDtypeStruct((B, 3, NB, _TH, W // 128, 8, 128),
                                 jnp.float32),
            jax.ShapeDtypeStruct((B, NB, _TH, W), jnp.int8),
        ],
        compiler_params=pltpu.CompilerParams(
            dimension_semantics=("parallel", "parallel", "parallel")),
    )(poses, depth, covariance, rotation, opacity, sh_color, confidence)

    fused = jnp.transpose(fused5.reshape(B, 3, P // 128, 8, 128),
                          (0, 2, 4, 1, 3)).reshape(B, P, 24)
    return fused, mask4.reshape(B, P).astype(jnp.bool_)
